# baseline (speedup 1.0000x reference)
import functools

import jax
import jax.numpy as jnp
from jax import lax
from jax.experimental import pallas as pl
from jax.experimental.pallas import tpu as pltpu
from jax.experimental.pallas import tpu_sc as plsc

N = 10000
E = 320000
D = 128

NC = 2
NS = 16
NW = NC * NS
EPT = E // NW
CH = 80
NCHUNK = EPT // CH
RPT = 624
ZR = 16


@functools.cache
def _sc_kernels():
    mesh = plsc.VectorSubcoreMesh(core_axis_name="c", subcore_axis_name="s")

    KD = 4
    NGD = (NCHUNK - 1) // KD

    @functools.partial(
        pl.kernel,
        mesh=mesh,
        out_type=jax.ShapeDtypeStruct((NC, N, 16), jnp.float32),
        scratch_types=[
            [pltpu.VMEM((CH,), jnp.int32) for _ in range(KD)],
            pltpu.VMEM((CH, 16), jnp.float32),
            pltpu.VMEM((ZR, 16), jnp.float32),
            pltpu.VMEM_SHARED((N, 16), jnp.float32),
            [pltpu.SemaphoreType.DMA for _ in range(KD)],
            [pltpu.SemaphoreType.DMA for _ in range(KD)],
        ],
    )
    def deg_kernel(dst_hbm, out_hbm, ia, ones_v, z_v, hist_sp,
                   sem_li, sem_s):
        c = lax.axis_index("c")
        s = lax.axis_index("s")
        wid = c * NS + s
        for i in range(CH):
            ones_v[i, :] = jnp.ones((16,), jnp.float32)
        for i in range(ZR):
            z_v[i, :] = jnp.zeros((16,), jnp.float32)

        def zero_body(k, carry):
            pltpu.sync_copy(z_v, hist_sp.at[pl.ds(s * RPT + k * ZR, ZR)])
            return carry

        lax.fori_loop(0, RPT // ZR, zero_body, 0)

        @pl.when(s == NS - 1)
        def _():
            pltpu.sync_copy(z_v, hist_sp.at[pl.ds(NS * RPT, ZR)])

        plsc.subcore_barrier()

        base = wid * EPT

        def ld(i, buf, sem):
            pltpu.async_copy(dst_hbm.at[pl.ds(base + i * CH, CH)], buf, sem)

        def ld_wait(i, buf, sem):
            pltpu.make_async_copy(dst_hbm.at[pl.ds(base + i * CH, CH)], buf,
                                  sem).wait()

        ld(0, ia[0], sem_li[0])

        def body(i, carry):
            ld(2 * i + 1, ia[1], sem_li[1])
            ld_wait(2 * i, ia[0], sem_li[0])
            pltpu.sync_copy(ones_v, hist_sp.at[ia[0]], add=True)
            ld(2 * i + 2, ia[0], sem_li[0])
            ld_wait(2 * i + 1, ia[1], sem_li[1])
            pltpu.sync_copy(ones_v, hist_sp.at[ia[1]], add=True)
            return carry

        lax.fori_loop(0, (NCHUNK - 1) // 2, body, 0)
        ld_wait(NCHUNK - 1, ia[0], sem_li[0])
        pltpu.sync_copy(ones_v, hist_sp.at[ia[0]], add=True)

        plsc.subcore_barrier()
        pltpu.sync_copy(hist_sp.at[pl.ds(s * RPT, RPT)],
                        out_hbm.at[c, pl.ds(s * RPT, RPT)])

        @pl.when(s == NS - 1)
        def _():
            pltpu.sync_copy(hist_sp.at[pl.ds(NS * RPT, ZR)],
                            out_hbm.at[c, pl.ds(NS * RPT, ZR)])

    KG = 4
    NG = (NCHUNK - 1) // KG

    @functools.partial(
        pl.kernel,
        mesh=mesh,
        out_type=jax.ShapeDtypeStruct((NC, N, D), jnp.float32),
        scratch_types=[
            [pltpu.VMEM((CH,), jnp.int32) for _ in range(KG)],
            [pltpu.VMEM((CH,), jnp.int32) for _ in range(KG)],
            [pltpu.VMEM((CH, D), jnp.float32) for _ in range(KG)],
            pltpu.VMEM((ZR, D), jnp.float32),
            pltpu.VMEM_SHARED((N, D), jnp.float32),
            [pltpu.SemaphoreType.DMA for _ in range(KG)],
            [pltpu.SemaphoreType.DMA for _ in range(KG)],
            [pltpu.SemaphoreType.DMA for _ in range(KG)],
            [pltpu.SemaphoreType.DMA for _ in range(KG)],
        ],
    )
    def scat_kernel(y_hbm, src_hbm, dst_hbm, out_hbm,
                    sa, da, rows, z_v, acc_sp, sem_g, sem_s, sem_ls, sem_ld):
        c = lax.axis_index("c")
        s = lax.axis_index("s")
        wid = c * NS + s
        base = wid * EPT
        for i in range(ZR):
            for j in range(D // 16):
                z_v[i, pl.ds(j * 16, 16)] = jnp.zeros((16,), jnp.float32)

        def zero_body(k, carry):
            pltpu.sync_copy(z_v, acc_sp.at[pl.ds(s * RPT + k * ZR, ZR)])
            return carry

        lax.fori_loop(0, RPT // ZR, zero_body, 0)

        @pl.when(s == NS - 1)
        def _():
            pltpu.sync_copy(z_v, acc_sp.at[pl.ds(NS * RPT, ZR)])

        plsc.subcore_barrier()

        def idx_load(i, buf, hbm, sem):
            pltpu.async_copy(hbm.at[pl.ds(base + i * CH, CH)], buf, sem)

        def idx_wait(i, buf, hbm, sem):
            pltpu.make_async_copy(hbm.at[pl.ds(base + i * CH, CH)], buf,
                                  sem).wait()

        for j in range(KG):
            idx_load(j, sa[j], src_hbm, sem_ls[j])
            idx_load(j, da[j], dst_hbm, sem_ld[j])

        def body(g, carry):
            gh = []
            for j in range(KG):
                ci = KG * g + j
                idx_wait(ci, sa[j], src_hbm, sem_ls[j])
                gh.append(
                    pltpu.async_copy(y_hbm.at[sa[j]], rows[j], sem_g[j]))
            sh = []
            for j in range(KG):
                ci = KG * g + j
                gh[j].wait()
                idx_wait(ci, da[j], dst_hbm, sem_ld[j])
                sh.append(
                    pltpu.async_copy(rows[j], acc_sp.at[da[j]], sem_s[j],
                                     add=True))
            for h in sh:
                h.wait()

            for j in range(KG):
                nxt = KG * (g + 1) + j

                @pl.when(nxt < NCHUNK)
                def _(nxt=nxt, j=j):
                    idx_load(nxt, sa[j], src_hbm, sem_ls[j])
                    idx_load(nxt, da[j], dst_hbm, sem_ld[j])

            return carry

        lax.fori_loop(0, NG, body, 0)
        ci = NCHUNK - 1
        idx_wait(ci, sa[0], src_hbm, sem_ls[0])
        pltpu.async_copy(y_hbm.at[sa[0]], rows[0], sem_g[0]).wait()
        idx_wait(ci, da[0], dst_hbm, sem_ld[0])
        pltpu.sync_copy(rows[0], acc_sp.at[da[0]], add=True)

        plsc.subcore_barrier()
        pltpu.sync_copy(acc_sp.at[pl.ds(s * RPT, RPT)],
                        out_hbm.at[c, pl.ds(s * RPT, RPT)])

        @pl.when(s == NS - 1)
        def _():
            pltpu.sync_copy(acc_sp.at[pl.ds(NS * RPT, ZR)],
                            out_hbm.at[c, pl.ds(NS * RPT, ZR)])

    return deg_kernel, scat_kernel


ROWS = 1000


def _dinv_from_hist(h_ref):
    deg = 1.0 + h_ref[0, :, :1] + h_ref[1, :, :1]
    return lax.rsqrt(deg)


def _pre_body(x_ref, w_ref, h_ref, y_ref):
    z = jnp.dot(x_ref[...], w_ref[...], preferred_element_type=jnp.float32)
    y_ref[...] = z * _dinv_from_hist(h_ref)


def _mid_body(s_ref, y_ref, h_ref, b_ref, m_ref, w_ref, o_ref):
    dinv = _dinv_from_hist(h_ref)
    agg = s_ref[0] + s_ref[1] + y_ref[...]
    x1 = 2.0 * m_ref[...] * jnp.maximum(dinv * agg + b_ref[...], 0.0)
    z = jnp.dot(x1, w_ref[...], preferred_element_type=jnp.float32)
    o_ref[...] = z * dinv


def _post_body(s_ref, y_ref, h_ref, b_ref, m_ref, o_ref):
    dinv = _dinv_from_hist(h_ref)
    agg = s_ref[0] + s_ref[1] + y_ref[...]
    o_ref[...] = 2.0 * m_ref[...] * jnp.maximum(dinv * agg + b_ref[...], 0.0)


def _row_spec():
    return pl.BlockSpec((ROWS, D), lambda i: (i, 0))


def _hist_spec():
    return pl.BlockSpec((NC, ROWS, 16), lambda i: (0, i, 0))


def _s_spec():
    return pl.BlockSpec((NC, ROWS, D), lambda i: (0, i, 0))


def _full_spec():
    return pl.BlockSpec((D, D), lambda i: (0, 0))


def _b_spec():
    return pl.BlockSpec((1, D), lambda i: (0, 0))


def _pre_call(x, W, h):
    return pl.pallas_call(
        _pre_body,
        grid=(N // ROWS,),
        in_specs=[_row_spec(), _full_spec(), _hist_spec()],
        out_specs=_row_spec(),
        out_shape=jax.ShapeDtypeStruct((N, D), jnp.float32),
    )(x, W, h)


def _mid_call(S, y, h, b, m, W):
    return pl.pallas_call(
        _mid_body,
        grid=(N // ROWS,),
        in_specs=[_s_spec(), _row_spec(), _hist_spec(), _b_spec(),
                  _row_spec(), _full_spec()],
        out_specs=_row_spec(),
        out_shape=jax.ShapeDtypeStruct((N, D), jnp.float32),
    )(S, y, h, b, m, W)


def _post_call(S, y, h, b, m):
    return pl.pallas_call(
        _post_body,
        grid=(N // ROWS,),
        in_specs=[_s_spec(), _row_spec(), _hist_spec(), _b_spec(),
                  _row_spec()],
        out_specs=_row_spec(),
        out_shape=jax.ShapeDtypeStruct((N, D), jnp.float32),
    )(S, y, h, b, m)


def kernel(tensor, edge_index, W1, b1, W2, b2):
    ei = edge_index.astype(jnp.int32)
    src = ei[0]
    dst = ei[1]
    m1 = jax.random.bernoulli(jax.random.key(1), 0.5, (N, D)).astype(jnp.float32)
    m2 = jax.random.bernoulli(jax.random.key(2), 0.5, (N, D)).astype(jnp.float32)

    deg_kernel, scat_kernel = _sc_kernels()
    h = deg_kernel(dst)
    y1 = _pre_call(tensor, W1, h)
    S1 = scat_kernel(y1, src, dst)
    y2 = _mid_call(S1, y1, h, b1.reshape(1, D), m1, W2)
    S2 = scat_kernel(y2, src, dst)
    out = _post_call(S2, y2, h, b2.reshape(1, D), m2)
    return out


# speedup vs baseline: 1.0250x; 1.0250x over previous
import functools

import jax
import jax.numpy as jnp
from jax import lax
from jax.experimental import pallas as pl
from jax.experimental.pallas import tpu as pltpu
from jax.experimental.pallas import tpu_sc as plsc

N = 10000
E = 320000
D = 128

NC = 2
NS = 16
NW = NC * NS
EPT = E // NW
CH = 80
NCHUNK = EPT // CH
RPT = 624
ZR = 16


@functools.cache
def _sc_kernels():
    mesh = plsc.VectorSubcoreMesh(core_axis_name="c", subcore_axis_name="s")

    KD = 4
    NGD = (NCHUNK - 1) // KD

    @functools.partial(
        pl.kernel,
        mesh=mesh,
        out_type=jax.ShapeDtypeStruct((NC, N, 16), jnp.float32),
        scratch_types=[
            [pltpu.VMEM((CH,), jnp.int32) for _ in range(KD)],
            [pltpu.VMEM((CH, 16), jnp.float32) for _ in range(KD)],
            pltpu.VMEM((ZR, 16), jnp.float32),
            pltpu.VMEM_SHARED((N, 16), jnp.float32),
            [pltpu.SemaphoreType.DMA for _ in range(KD)],
            [pltpu.SemaphoreType.DMA for _ in range(KD)],
        ],
    )
    def deg_kernel(dst_hbm, out_hbm, ia, ones, z_v, hist_sp,
                   sem_li, sem_s):
        c = lax.axis_index("c")
        s = lax.axis_index("s")
        wid = c * NS + s
        for j in range(KD):
            for i in range(CH):
                ones[j][i, :] = jnp.ones((16,), jnp.float32)
        for i in range(ZR):
            z_v[i, :] = jnp.zeros((16,), jnp.float32)

        def zero_body(k, carry):
            pltpu.sync_copy(z_v, hist_sp.at[pl.ds(s * RPT + k * ZR, ZR)])
            return carry

        lax.fori_loop(0, RPT // ZR, zero_body, 0)

        @pl.when(s == NS - 1)
        def _():
            pltpu.sync_copy(z_v, hist_sp.at[pl.ds(NS * RPT, ZR)])

        plsc.subcore_barrier()

        base = wid * EPT

        def ld(i, buf, sem):
            pltpu.async_copy(dst_hbm.at[pl.ds(base + i * CH, CH)], buf, sem)

        def ld_wait(i, buf, sem):
            pltpu.make_async_copy(dst_hbm.at[pl.ds(base + i * CH, CH)], buf,
                                  sem).wait()

        for j in range(KD):
            ld(j, ia[j], sem_li[j])

        def body(g, carry):
            sh = []
            for j in range(KD):
                ld_wait(KD * g + j, ia[j], sem_li[j])
                sh.append(
                    pltpu.async_copy(ones[j], hist_sp.at[ia[j]], sem_s[j],
                                     add=True))
            for h in sh:
                h.wait()
            for j in range(KD):
                nxt = KD * (g + 1) + j

                @pl.when(nxt < NCHUNK)
                def _(nxt=nxt, j=j):
                    ld(nxt, ia[j], sem_li[j])

            return carry

        lax.fori_loop(0, NGD, body, 0)
        ld_wait(NCHUNK - 1, ia[0], sem_li[0])
        pltpu.sync_copy(ones[0], hist_sp.at[ia[0]], add=True)

        plsc.subcore_barrier()
        pltpu.sync_copy(hist_sp.at[pl.ds(s * RPT, RPT)],
                        out_hbm.at[c, pl.ds(s * RPT, RPT)])

        @pl.when(s == NS - 1)
        def _():
            pltpu.sync_copy(hist_sp.at[pl.ds(NS * RPT, ZR)],
                            out_hbm.at[c, pl.ds(NS * RPT, ZR)])

    KG = 4
    NG = (NCHUNK - 1) // KG

    @functools.partial(
        pl.kernel,
        mesh=mesh,
        out_type=jax.ShapeDtypeStruct((NC, N, D), jnp.float32),
        scratch_types=[
            [pltpu.VMEM((CH,), jnp.int32) for _ in range(KG)],
            [pltpu.VMEM((CH,), jnp.int32) for _ in range(KG)],
            [pltpu.VMEM((CH, D), jnp.float32) for _ in range(KG)],
            pltpu.VMEM((ZR, D), jnp.float32),
            pltpu.VMEM_SHARED((N, D), jnp.float32),
            [pltpu.SemaphoreType.DMA for _ in range(KG)],
            [pltpu.SemaphoreType.DMA for _ in range(KG)],
            [pltpu.SemaphoreType.DMA for _ in range(KG)],
            [pltpu.SemaphoreType.DMA for _ in range(KG)],
        ],
    )
    def scat_kernel(y_hbm, src_hbm, dst_hbm, out_hbm,
                    sa, da, rows, z_v, acc_sp, sem_g, sem_s, sem_ls, sem_ld):
        c = lax.axis_index("c")
        s = lax.axis_index("s")
        wid = c * NS + s
        base = wid * EPT
        for i in range(ZR):
            for j in range(D // 16):
                z_v[i, pl.ds(j * 16, 16)] = jnp.zeros((16,), jnp.float32)

        def zero_body(k, carry):
            pltpu.sync_copy(z_v, acc_sp.at[pl.ds(s * RPT + k * ZR, ZR)])
            return carry

        lax.fori_loop(0, RPT // ZR, zero_body, 0)

        @pl.when(s == NS - 1)
        def _():
            pltpu.sync_copy(z_v, acc_sp.at[pl.ds(NS * RPT, ZR)])

        plsc.subcore_barrier()

        def idx_load(i, buf, hbm, sem):
            pltpu.async_copy(hbm.at[pl.ds(base + i * CH, CH)], buf, sem)

        def idx_wait(i, buf, hbm, sem):
            pltpu.make_async_copy(hbm.at[pl.ds(base + i * CH, CH)], buf,
                                  sem).wait()

        for j in range(KG):
            idx_load(j, sa[j], src_hbm, sem_ls[j])
            idx_load(j, da[j], dst_hbm, sem_ld[j])

        def body(g, carry):
            gh = []
            for j in range(KG):
                ci = KG * g + j
                idx_wait(ci, sa[j], src_hbm, sem_ls[j])
                gh.append(
                    pltpu.async_copy(y_hbm.at[sa[j]], rows[j], sem_g[j]))
            sh = []
            for j in range(KG):
                ci = KG * g + j
                gh[j].wait()
                idx_wait(ci, da[j], dst_hbm, sem_ld[j])
                sh.append(
                    pltpu.async_copy(rows[j], acc_sp.at[da[j]], sem_s[j],
                                     add=True))
            for h in sh:
                h.wait()

            for j in range(KG):
                nxt = KG * (g + 1) + j

                @pl.when(nxt < NCHUNK)
                def _(nxt=nxt, j=j):
                    idx_load(nxt, sa[j], src_hbm, sem_ls[j])
                    idx_load(nxt, da[j], dst_hbm, sem_ld[j])

            return carry

        lax.fori_loop(0, NG, body, 0)
        ci = NCHUNK - 1
        idx_wait(ci, sa[0], src_hbm, sem_ls[0])
        pltpu.async_copy(y_hbm.at[sa[0]], rows[0], sem_g[0]).wait()
        idx_wait(ci, da[0], dst_hbm, sem_ld[0])
        pltpu.sync_copy(rows[0], acc_sp.at[da[0]], add=True)

        plsc.subcore_barrier()
        pltpu.sync_copy(acc_sp.at[pl.ds(s * RPT, RPT)],
                        out_hbm.at[c, pl.ds(s * RPT, RPT)])

        @pl.when(s == NS - 1)
        def _():
            pltpu.sync_copy(acc_sp.at[pl.ds(NS * RPT, ZR)],
                            out_hbm.at[c, pl.ds(NS * RPT, ZR)])

    return deg_kernel, scat_kernel


ROWS = 1000


def _dinv_from_hist(h_ref):
    deg = 1.0 + h_ref[0, :, :1] + h_ref[1, :, :1]
    return lax.rsqrt(deg)


def _pre_body(x_ref, w_ref, h_ref, y_ref):
    z = jnp.dot(x_ref[...], w_ref[...], preferred_element_type=jnp.float32)
    y_ref[...] = z * _dinv_from_hist(h_ref)


def _mid_body(s_ref, y_ref, h_ref, b_ref, m_ref, w_ref, o_ref):
    dinv = _dinv_from_hist(h_ref)
    agg = s_ref[0] + s_ref[1] + y_ref[...]
    x1 = 2.0 * m_ref[...] * jnp.maximum(dinv * agg + b_ref[...], 0.0)
    z = jnp.dot(x1, w_ref[...], preferred_element_type=jnp.float32)
    o_ref[...] = z * dinv


def _post_body(s_ref, y_ref, h_ref, b_ref, m_ref, o_ref):
    dinv = _dinv_from_hist(h_ref)
    agg = s_ref[0] + s_ref[1] + y_ref[...]
    o_ref[...] = 2.0 * m_ref[...] * jnp.maximum(dinv * agg + b_ref[...], 0.0)


def _row_spec():
    return pl.BlockSpec((ROWS, D), lambda i: (i, 0))


def _hist_spec():
    return pl.BlockSpec((NC, ROWS, 16), lambda i: (0, i, 0))


def _s_spec():
    return pl.BlockSpec((NC, ROWS, D), lambda i: (0, i, 0))


def _full_spec():
    return pl.BlockSpec((D, D), lambda i: (0, 0))


def _b_spec():
    return pl.BlockSpec((1, D), lambda i: (0, 0))


def _pre_call(x, W, h):
    return pl.pallas_call(
        _pre_body,
        grid=(N // ROWS,),
        in_specs=[_row_spec(), _full_spec(), _hist_spec()],
        out_specs=_row_spec(),
        out_shape=jax.ShapeDtypeStruct((N, D), jnp.float32),
    )(x, W, h)


def _mid_call(S, y, h, b, m, W):
    return pl.pallas_call(
        _mid_body,
        grid=(N // ROWS,),
        in_specs=[_s_spec(), _row_spec(), _hist_spec(), _b_spec(),
                  _row_spec(), _full_spec()],
        out_specs=_row_spec(),
        out_shape=jax.ShapeDtypeStruct((N, D), jnp.float32),
    )(S, y, h, b, m, W)


def _post_call(S, y, h, b, m):
    return pl.pallas_call(
        _post_body,
        grid=(N // ROWS,),
        in_specs=[_s_spec(), _row_spec(), _hist_spec(), _b_spec(),
                  _row_spec()],
        out_specs=_row_spec(),
        out_shape=jax.ShapeDtypeStruct((N, D), jnp.float32),
    )(S, y, h, b, m)


def kernel(tensor, edge_index, W1, b1, W2, b2):
    ei = edge_index.astype(jnp.int32)
    src = ei[0]
    dst = ei[1]
    m1 = jax.random.bernoulli(jax.random.key(1), 0.5, (N, D)).astype(jnp.float32)
    m2 = jax.random.bernoulli(jax.random.key(2), 0.5, (N, D)).astype(jnp.float32)

    deg_kernel, scat_kernel = _sc_kernels()
    h = deg_kernel(dst)
    y1 = _pre_call(tensor, W1, h)
    S1 = scat_kernel(y1, src, dst)
    y2 = _mid_call(S1, y1, h, b1.reshape(1, D), m1, W2)
    S2 = scat_kernel(y2, src, dst)
    out = _post_call(S2, y2, h, b2.reshape(1, D), m2)
    return out
